# baseline (speedup 1.0000x reference)
import jax
import jax.numpy as jnp
from jax import lax
from jax.experimental import pallas as pl
from jax.experimental.pallas import tpu as pltpu
from jax.experimental.pallas import tpu_sc as plsc

_NUM_ATOMS = 320000
_FEA = 128
_NUM_SEG = 10000
_NC = 2
_NS = 16
_APT = _NUM_ATOMS // (_NC * _NS)
_CHUNK = 80
_NCHUNK = _APT // _CHUNK
_ROWS_PT = _NUM_SEG // _NS
_ZROWS = 25


def _body(feas, owner2, out, rb0, rb1, rb2,
          obig, zbuf, acc, bsem,
          lsem0, lsem1, lsem2, ssem0, ssem1, ssem2):
    c = lax.axis_index("c")
    s = lax.axis_index("s")
    base = (c * _NS + s) * _APT
    cbase = (c * _NS + s) * _NCHUNK
    rb = (rb0, rb1, rb2)
    lsem = (lsem0, lsem1, lsem2)
    ssem = (ssem0, ssem1, ssem2)

    def _rows_at(a0):
        return feas.at[pl.ds(a0, _CHUNK), :]

    def _issue_load(k, b):
        pltpu.async_copy(_rows_at(base + k * _CHUNK), rb[b], lsem[b])

    def _wait_load(b):
        pltpu.make_async_copy(_rows_at(base), rb[b], lsem[b]).wait()

    def _start_scatter(k, b):
        pltpu.make_async_copy(
            rb[b], acc.at[obig.at[k]], ssem[b]).start(add=True)

    def _wait_scatter(b):
        pltpu.make_async_copy(rb[b], acc.at[obig.at[0]], ssem[b]).wait()

    pltpu.async_copy(owner2.at[pl.ds(cbase, _NCHUNK), :], obig, bsem)
    for b in range(2):
        _issue_load(b, b)

    def _zrow(i, carry):
        for j in range(_FEA // 16):
            zbuf[i, pl.ds(j * 16, 16)] = jnp.zeros((16,), jnp.float32)
        return carry

    lax.fori_loop(0, _ZROWS, _zrow, 0)
    r0 = s * _ROWS_PT
    for j in range(_ROWS_PT // _ZROWS):
        pltpu.sync_copy(zbuf, acc.at[pl.ds(r0 + j * _ZROWS, _ZROWS), :])
    pltpu.make_async_copy(
        owner2.at[pl.ds(cbase, _NCHUNK), :], obig, bsem).wait()
    plsc.subcore_barrier()

    _wait_load(0)
    _start_scatter(0, 0)
    _issue_load(2, 2)
    _wait_load(1)
    _start_scatter(1, 1)
    _wait_scatter(0)
    _issue_load(3, 0)

    def _tri(i, carry):
        for j in range(3):
            k = 2 + 3 * i + j
            b = (2 + j) % 3
            _wait_load(b)
            _start_scatter(k, b)
            bn = (1 + j) % 3
            _wait_scatter(bn)
            kn = jnp.minimum(k + 2, _NCHUNK - 1)
            _issue_load(kn, bn)
        return carry

    lax.fori_loop(0, (_NCHUNK - 2) // 3, _tri, 0)
    for b in (0, 2):
        _wait_load(b)
    _wait_scatter(1)
    plsc.subcore_barrier()

    pltpu.sync_copy(
        acc.at[pl.ds(r0, _ROWS_PT), :],
        out.at[c, pl.ds(r0, _ROWS_PT), :],
    )


_pool = pl.kernel(
    _body,
    out_type=jax.ShapeDtypeStruct((_NC, _NUM_SEG, _FEA), jnp.float32),
    mesh=plsc.VectorSubcoreMesh(
        core_axis_name="c", subcore_axis_name="s", num_cores=_NC,
        num_subcores=_NS,
    ),
    scratch_types=(
        [pltpu.VMEM((_CHUNK, _FEA), jnp.float32)] * 3
        + [
            pltpu.VMEM((_NCHUNK, _CHUNK), jnp.int32),
            pltpu.VMEM((_ZROWS, _FEA), jnp.float32),
            pltpu.VMEM_SHARED((_NUM_SEG, _FEA), jnp.float32),
        ]
        + [pltpu.SemaphoreType.DMA] * 7
    ),
    compiler_params=pltpu.CompilerParams(use_tc_tiling_on_sc=False),
)


def _combine_body(p_ref, o_ref):
    o_ref[...] = p_ref[0] + p_ref[1]


_combine = pl.pallas_call(
    _combine_body,
    grid=(10,),
    in_specs=[pl.BlockSpec((2, _NUM_SEG // 10, _FEA), lambda i: (0, i, 0))],
    out_specs=pl.BlockSpec((_NUM_SEG // 10, _FEA), lambda i: (i, 0)),
    out_shape=jax.ShapeDtypeStruct((_NUM_SEG, _FEA), jnp.float32),
)


@jax.jit
def kernel(atom_feas, atom_owner):
    owner2 = atom_owner.astype(jnp.int32).reshape(
        _NUM_ATOMS // _CHUNK, _CHUNK)
    partial = _pool(atom_feas, owner2)
    return _combine(partial)


# speedup vs baseline: 1.2014x; 1.2014x over previous
import jax
import jax.numpy as jnp
from jax import lax
from jax.experimental import pallas as pl
from jax.experimental.pallas import tpu as pltpu
from jax.experimental.pallas import tpu_sc as plsc

_NUM_ATOMS = 320000
_FEA = 128
_NUM_SEG = 10000
_NC = 2
_NS = 16
_COLS = _FEA // _NC
_APT = _NUM_ATOMS // _NS
_CHUNK = 80
_NCHUNK = _APT // _CHUNK
_ROWS_PT = _NUM_SEG // _NS
_ZROWS = 125


def _body(feas, owner2, out, *refs):
    rb = refs[:12]
    obig, zbuf, acc, bsem = refs[12:16]
    lsem = refs[16:28]
    ssem = refs[28:40]
    c = lax.axis_index("c")
    s = lax.axis_index("s")
    col0 = c * _COLS
    base = s * _APT

    def _rows_at(a0):
        return feas.at[pl.ds(a0, _CHUNK), pl.ds(col0, _COLS)]

    def _issue_load(k, b):
        pltpu.async_copy(_rows_at(base + k * _CHUNK), rb[b], lsem[b])

    def _wait_load(b):
        pltpu.make_async_copy(_rows_at(base), rb[b], lsem[b]).wait()

    def _start_scatter(k, b):
        pltpu.make_async_copy(
            rb[b], acc.at[obig.at[k]], ssem[b]).start(add=True)

    def _wait_scatter(b):
        pltpu.make_async_copy(rb[b], acc.at[obig.at[0]], ssem[b]).wait()

    pltpu.async_copy(owner2.at[pl.ds(s * _NCHUNK, _NCHUNK), :], obig, bsem)
    for b in range(10):
        _issue_load(b, b)

    def _zrow(i, carry):
        for j in range(_COLS // 16):
            zbuf[i, pl.ds(j * 16, 16)] = jnp.zeros((16,), jnp.float32)
        return carry

    lax.fori_loop(0, _ZROWS, _zrow, 0)
    r0 = s * _ROWS_PT
    for j in range(_ROWS_PT // _ZROWS):
        pltpu.sync_copy(zbuf, acc.at[pl.ds(r0 + j * _ZROWS, _ZROWS), :])
    pltpu.make_async_copy(
        owner2.at[pl.ds(s * _NCHUNK, _NCHUNK), :], obig, bsem).wait()
    plsc.subcore_barrier()

    for k in range(2):
        _wait_load(k)
        _start_scatter(k, k)
        _issue_load(k + 10, k + 10)

    def _twelve(i, carry):
        for j in range(12):
            k = 2 + 12 * i + j
            b = (2 + j) % 12
            _wait_load(b)
            _start_scatter(k, b)
            bn = j
            _wait_scatter(bn)
            kn = jnp.minimum(k + 10, _NCHUNK - 1)
            _issue_load(kn, bn)
        return carry

    lax.fori_loop(0, (_NCHUNK - 10) // 12, _twelve, 0)
    for k in range(_NCHUNK - 8, _NCHUNK):
        b = k % 12
        _wait_load(b)
        _start_scatter(k, b)
        _wait_scatter((k + 10) % 12)
        _issue_load(_NCHUNK - 1, (k + 10) % 12)
    for b in list(range(8)) + [10, 11]:
        _wait_load(b)
    for b in (8, 9):
        _wait_scatter(b)
    plsc.subcore_barrier()

    pltpu.sync_copy(
        acc.at[pl.ds(r0, _ROWS_PT), :],
        out.at[pl.ds(r0, _ROWS_PT), pl.ds(col0, _COLS)],
    )


_pool = pl.kernel(
    _body,
    out_type=jax.ShapeDtypeStruct((_NUM_SEG, _FEA), jnp.float32),
    mesh=plsc.VectorSubcoreMesh(
        core_axis_name="c", subcore_axis_name="s", num_cores=_NC,
        num_subcores=_NS,
    ),
    scratch_types=(
        [pltpu.VMEM((_CHUNK, _COLS), jnp.float32)] * 12
        + [
            pltpu.VMEM((_NCHUNK, _CHUNK), jnp.int32),
            pltpu.VMEM((_ZROWS, _COLS), jnp.float32),
            pltpu.VMEM_SHARED((_NUM_SEG, _COLS), jnp.float32),
        ]
        + [pltpu.SemaphoreType.DMA] * 25
    ),
    compiler_params=pltpu.CompilerParams(use_tc_tiling_on_sc=False),
)


@jax.jit
def kernel(atom_feas, atom_owner):
    owner2 = atom_owner.astype(jnp.int32).reshape(
        _NUM_ATOMS // _CHUNK, _CHUNK)
    return _pool(atom_feas, owner2)


# speedup vs baseline: 1.2150x; 1.0113x over previous
import jax
import jax.numpy as jnp
from jax import lax
from jax.experimental import pallas as pl
from jax.experimental.pallas import tpu as pltpu
from jax.experimental.pallas import tpu_sc as plsc

_NUM_ATOMS = 320000
_FEA = 128
_NUM_SEG = 10000
_NC = 2
_NS = 16
_COLS = _FEA // _NC
_APT = _NUM_ATOMS // _NS
_CHUNK = 80
_NCHUNK = _APT // _CHUNK
_ROWS_PT = _NUM_SEG // _NS
_ZROWS = 125


def _body(feas, owner2, out, rb0, rb1, rb2, rb3, rb4, rb5, rb6, rb7,
          obig, zbuf, acc, bsem,
          lsem0, lsem1, lsem2, lsem3, lsem4, lsem5, lsem6, lsem7,
          ssem0, ssem1, ssem2, ssem3, ssem4, ssem5, ssem6, ssem7):
    c = lax.axis_index("c")
    s = lax.axis_index("s")
    col0 = c * _COLS
    base = s * _APT
    rb = (rb0, rb1, rb2, rb3, rb4, rb5, rb6, rb7)
    lsem = (lsem0, lsem1, lsem2, lsem3, lsem4, lsem5, lsem6, lsem7)
    ssem = (ssem0, ssem1, ssem2, ssem3, ssem4, ssem5, ssem6, ssem7)

    def _rows_at(a0):
        return feas.at[pl.ds(a0, _CHUNK), pl.ds(col0, _COLS)]

    def _issue_load(k, b):
        pltpu.async_copy(_rows_at(base + k * _CHUNK), rb[b], lsem[b])

    def _wait_load(b):
        pltpu.make_async_copy(_rows_at(base), rb[b], lsem[b]).wait()

    def _start_scatter(k, b):
        pltpu.make_async_copy(
            rb[b], acc.at[obig.at[k]], ssem[b]).start(add=True)

    def _wait_scatter(b):
        pltpu.make_async_copy(rb[b], acc.at[obig.at[0]], ssem[b]).wait()

    pltpu.async_copy(owner2.at[pl.ds(s * _NCHUNK, _NCHUNK), :], obig, bsem)
    for b in range(6):
        _issue_load(b, b)

    def _zrow(i, carry):
        for j in range(_COLS // 16):
            zbuf[i, pl.ds(j * 16, 16)] = jnp.zeros((16,), jnp.float32)
        return carry

    lax.fori_loop(0, _ZROWS, _zrow, 0)
    r0 = s * _ROWS_PT
    for j in range(_ROWS_PT // _ZROWS):
        pltpu.async_copy(
            zbuf, acc.at[pl.ds(r0 + j * _ZROWS, _ZROWS), :], ssem[j])
    for j in range(_ROWS_PT // _ZROWS):
        pltpu.make_async_copy(
            zbuf, acc.at[pl.ds(r0 + j * _ZROWS, _ZROWS), :], ssem[j]).wait()
    pltpu.make_async_copy(
        owner2.at[pl.ds(s * _NCHUNK, _NCHUNK), :], obig, bsem).wait()
    plsc.subcore_barrier()

    for k in range(2):
        _wait_load(k)
        _start_scatter(k, k)
        _issue_load(k + 6, k + 6)

    def _oct(i, carry):
        for j in range(8):
            k = 2 + 8 * i + j
            b = (2 + j) % 8
            _wait_load(b)
            _start_scatter(k, b)
            bn = j
            _wait_scatter(bn)
            kn = jnp.minimum(k + 6, _NCHUNK - 1)
            _issue_load(kn, bn)
        return carry

    lax.fori_loop(0, (_NCHUNK - 2) // 8, _oct, 0)
    for b in range(6):
        _wait_load(b + 2)
    for b in range(2):
        _wait_scatter(b)
    plsc.subcore_barrier()

    pltpu.sync_copy(
        acc.at[pl.ds(r0, _ROWS_PT), :],
        out.at[pl.ds(r0, _ROWS_PT), pl.ds(col0, _COLS)],
    )


_pool = pl.kernel(
    _body,
    out_type=jax.ShapeDtypeStruct((_NUM_SEG, _FEA), jnp.float32),
    mesh=plsc.VectorSubcoreMesh(
        core_axis_name="c", subcore_axis_name="s", num_cores=_NC,
        num_subcores=_NS,
    ),
    scratch_types=(
        [pltpu.VMEM((_CHUNK, _COLS), jnp.float32)] * 8
        + [
            pltpu.VMEM((_NCHUNK, _CHUNK), jnp.int32),
            pltpu.VMEM((_ZROWS, _COLS), jnp.float32),
            pltpu.VMEM_SHARED((_NUM_SEG, _COLS), jnp.float32),
        ]
        + [pltpu.SemaphoreType.DMA] * 17
    ),
    compiler_params=pltpu.CompilerParams(use_tc_tiling_on_sc=False),
)


@jax.jit
def kernel(atom_feas, atom_owner):
    owner2 = atom_owner.astype(jnp.int32).reshape(
        _NUM_ATOMS // _CHUNK, _CHUNK)
    return _pool(atom_feas, owner2)
